# baseline (speedup 1.0000x reference)
import functools

import jax
import jax.numpy as jnp
from jax import lax
from jax.experimental import pallas as pl
from jax.experimental.pallas import tpu as pltpu
from jax.experimental.pallas import tpu_sc as plsc

_N_ROWS = 16384
_N_COLS = 1024
_L = 16
_NVREG = _N_COLS // _L
_R = 16
_NBUF = 4
_G = 4


def _scan_block(buf, slot):
    last = jnp.full((_L, 1), _L - 1, dtype=jnp.int32)
    dnums = lax.GatherDimensionNumbers(
        offset_dims=(), collapsed_slice_dims=(0,), start_index_map=(0,))

    def bcast_last(s):
        return lax.gather(s, last, dnums, slice_sizes=(1,),
                          mode=lax.GatherScatterMode.PROMISE_IN_BOUNDS)

    for r0 in range(0, _R, _G):
        def body(i, carries):
            new = []
            for g in range(_G):
                v = buf[slot, r0 + g, pl.ds(i * _L, _L)]
                s = plsc.cumsum(v)
                buf[slot, r0 + g, pl.ds(i * _L, _L)] = s - v + carries[g]
                new.append(carries[g] + bcast_last(s))
            return tuple(new)

        lax.fori_loop(
            0, _NVREG, body,
            tuple(jnp.zeros((_L,), jnp.float32) for _ in range(_G)),
            unroll=False)


def _make_kernel():
    info = plsc.get_sparse_core_info()
    nw = info.num_cores * info.num_subcores
    rows_per_worker = _N_ROWS // nw
    nblk = rows_per_worker // _R

    mesh = plsc.VectorSubcoreMesh(core_axis_name="c", subcore_axis_name="s")

    @functools.partial(
        pl.kernel,
        mesh=mesh,
        out_type=jax.ShapeDtypeStruct((_N_ROWS, _N_COLS), jnp.float32),
        scratch_types=[
            pltpu.VMEM((_NBUF, _R, _N_COLS), jnp.float32),
            pltpu.SemaphoreType.DMA((_NBUF,)),
            pltpu.SemaphoreType.DMA((_NBUF,)),
        ],
        compiler_params=pltpu.CompilerParams(needs_layout_passes=False),
    )
    def scan_all(x_hbm, out_hbm, buf, in_sems, out_sems):
        wid = lax.axis_index("s") * info.num_cores + lax.axis_index("c")
        base = wid * rows_per_worker

        def in_copy(b, slot):
            return pltpu.make_async_copy(
                x_hbm.at[pl.ds(base + b * _R, _R)], buf.at[slot],
                in_sems.at[slot])

        def out_copy(b, slot):
            return pltpu.make_async_copy(
                buf.at[slot], out_hbm.at[pl.ds(base + b * _R, _R)],
                out_sems.at[slot])

        for b in range(_NBUF - 1):
            in_copy(b, b).start()

        def step(g, _):
            for k in range(_NBUF):
                b = g * _NBUF + k
                in_copy(b, k).wait()
                _scan_block(buf, k)
                out_copy(b, k).start()

                nk = (k + _NBUF - 1) % _NBUF

                @pl.when(b + _NBUF - 1 < nblk)
                def _():
                    @pl.when(b >= 1)
                    def _():
                        out_copy(b - 1, nk).wait()
                    in_copy(b + _NBUF - 1, nk).start()
            return 0

        lax.fori_loop(0, nblk // _NBUF, step, 0, unroll=False)

        for b in range(nblk - _NBUF, nblk):
            out_copy(b, b % _NBUF).wait()

    return scan_all


_scan_all = _make_kernel()


def kernel(x):
    return _scan_all(x)
